# baseline (speedup 1.0000x reference)
import jax
import jax.numpy as jnp
from jax import lax
from jax.experimental import pallas as pl
from jax.experimental.pallas import tpu as pltpu
from jax.experimental.pallas import tpu_sc as plsc

_HU = (-1000.0, -900.0, -400.0, -100.0, -50.0, -10.0,
       20.0, 40.0, 60.0, 100.0, 800.0, 1000.0)
_NC = 2
_NS = 16
_L = 16
_P = 4
_G = 24
_W = 96
_VD = 8
_NV = 13
_CH = _P * _P * _P * _VD
_NPAIR = _G * _CH // _L
_UNITS = 2 * _G * _G
_UPW = _UNITS // (_NC * _NS)


def _bin16(xv):
    b = jnp.where(xv >= _HU[0], 1, 0)
    for t in _HU[1:]:
        b = b + jnp.where(xv >= t, 1, 0)
    return b


def _tec_body(x_hbm, vtp_hbm, out_hbm,
              xb_v, bp_v, g_v, out_v, vtp_sh,
              sem_g0, sem_g1, sem_o0, sem_o1):
    wid = lax.axis_index("s") * _NC + lax.axis_index("c")
    sem_g = (sem_g0, sem_g1)
    sem_o = (sem_o0, sem_o1)

    @pl.when(lax.axis_index("s") == 0)
    def _():
        pltpu.sync_copy(vtp_hbm, vtp_sh)

    plsc.subcore_barrier()

    lane = lax.iota(jnp.int32, _L)
    pat_e = (2 * lane) % _L
    pat_o = (2 * lane + 1) % _L
    low8 = lane < 8

    def unit_coords(u):
        b = u // (_G * _G)
        r = u % (_G * _G)
        return b, r // _G, r % _G

    def phase1(u, buf):
        b, dp, hp = unit_coords(u)
        pltpu.sync_copy(x_hbm.at[b, dp, :, hp, :, :], xb_v)

        def do_row(pdph, c2):
            bv = [_bin16(xb_v[pdph // 4, pdph % 4, pl.ds(q * _L, _L)])
                  for q in range(6)]
            for m in range(3):
                b1, b2 = bv[2 * m], bv[2 * m + 1]
                ev = jnp.where(low8, jnp.take_along_axis(b1, pat_e, axis=0),
                               jnp.take_along_axis(b2, pat_e, axis=0))
                od = jnp.where(low8, jnp.take_along_axis(b1, pat_o, axis=0),
                               jnp.take_along_axis(b2, pat_o, axis=0))
                bp_v[buf, pl.ds(pdph * 48 + m * _L, _L)] = ev * _L + od
            return c2

        lax.fori_loop(0, _P * _P, do_row, 0)

    def fire_gathers(buf):
        for q in range(_NPAIR // 128):
            pltpu.async_copy(
                vtp_sh.at[bp_v.at[buf, pl.ds(128 * q, 128)]],
                g_v.at[buf, pl.ds(128 * q, 128)],
                sem_g[buf])

    def wait_gathers(buf):
        for q in range(_NPAIR // 128):
            pltpu.make_async_copy(
                vtp_sh.at[bp_v.at[buf, pl.ds(128 * q, 128)]],
                g_v.at[buf, pl.ds(128 * q, 128)],
                sem_g[buf]).wait()

    def emit(u, buf):
        b, dp, hp = unit_coords(u)
        wait_gathers(buf)

        def do_reorder(pdph, c3):
            for m in range(3):
                for l in range(_L):
                    p = m * _L + l
                    wp, j = p // 2, p % 2
                    out_v[buf, wp, pl.ds(pdph * 2 * _L + j * _L, _L)] = (
                        g_v[buf, pdph * 48 + p])
            return c3

        lax.fori_loop(0, _P * _P, do_reorder, 0)
        pltpu.async_copy(out_v.at[buf], out_hbm.at[b, dp, hp], sem_o[buf])

    def wait_out(u, buf):
        b, dp, hp = unit_coords(u)
        pltpu.make_async_copy(out_v.at[buf], out_hbm.at[b, dp, hp],
                              sem_o[buf]).wait()

    u0 = wid * _UPW

    def do_pair_of_units(i, carry):
        ua = u0 + 2 * i

        @pl.when(i > 0)
        def _():
            wait_out(ua - 2, 0)

        phase1(ua, 0)
        fire_gathers(0)

        @pl.when(i > 0)
        def _():
            emit(ua - 1, 1)

        phase1(ua + 1, 1)
        fire_gathers(1)
        emit(ua, 0)
        return carry

    lax.fori_loop(0, _UPW // 2, do_pair_of_units, 0)
    emit(u0 + _UPW - 1, 1)
    wait_out(u0 + _UPW - 2, 0)
    wait_out(u0 + _UPW - 1, 1)


def _make_sc_call():
    mesh = plsc.VectorSubcoreMesh(core_axis_name="c", subcore_axis_name="s",
                                  num_cores=_NC, num_subcores=_NS)
    return pl.kernel(
        _tec_body,
        out_type=jax.ShapeDtypeStruct((2, _G, _G, _G, _CH), jnp.float32),
        mesh=mesh,
        compiler_params=pltpu.CompilerParams(use_tc_tiling_on_sc=False),
        scratch_types=[
            pltpu.VMEM((_P, _P, _W), jnp.float32),
            pltpu.VMEM((2, _NPAIR), jnp.int32),
            pltpu.VMEM((2, _NPAIR, _L), jnp.float32),
            pltpu.VMEM((2, _G, _CH), jnp.float32),
            pltpu.VMEM_SHARED((_NV * _L, _L), jnp.float32),
            pltpu.SemaphoreType.DMA,
            pltpu.SemaphoreType.DMA,
            pltpu.SemaphoreType.DMA,
            pltpu.SemaphoreType.DMA,
        ],
    )


@jax.jit
def kernel(x, vectors):
    B, C, D, H, W = x.shape
    x6 = x.reshape(B, D // _P, _P, H // _P, _P, W)
    va = jnp.broadcast_to(vectors[:, None, :], (_NV, _NV, _VD))
    vb = jnp.broadcast_to(vectors[None, :, :], (_NV, _NV, _VD))
    vtp = jnp.concatenate([va, vb], axis=-1)
    vtp = jnp.pad(vtp, ((0, 0), (0, _L - _NV), (0, 0)))
    vtp = vtp.reshape(_NV * _L, _L)
    out = _make_sc_call()(x6, vtp)
    return jnp.transpose(out, (0, 4, 1, 2, 3))


# speedup vs baseline: 1.8372x; 1.8372x over previous
import jax
import jax.numpy as jnp
from jax import lax
from jax.experimental import pallas as pl
from jax.experimental.pallas import tpu as pltpu
from jax.experimental.pallas import tpu_sc as plsc

_HU = (-1000.0, -900.0, -400.0, -100.0, -50.0, -10.0,
       20.0, 40.0, 60.0, 100.0, 800.0, 1000.0)
_NC = 2
_NS = 16
_L = 16
_P = 4
_G = 24
_W = 96
_VD = 8
_NV = 13
_CH = _P * _P * _P * _VD
_UPW = 2 * _G * _G // (_NC * _NS)


def _bin16(xv):
    b = jnp.where(xv >= _HU[0], 1, 0)
    for t in _HU[1:]:
        b = b + jnp.where(xv >= t, 1, 0)
    return b


def _coords(u):
    b = u // (_G * _G)
    r = u % (_G * _G)
    return b, r // _G, r % _G


def _tec_body(x_hbm, vtp_hbm, out_hbm,
              xb_v, vtp_v, bp_v, out_v, sem_out0, sem_out1):
    wid = lax.axis_index("s") * _NC + lax.axis_index("c")
    sem_out = (sem_out0, sem_out1)

    pltpu.sync_copy(vtp_hbm, vtp_v)

    lane = lax.iota(jnp.int32, _L)
    pat_e = (2 * lane) % _L
    pat_o = (2 * lane + 1) % _L
    lo = jnp.where(lane < 8, 1, 0)

    def phase1(u):
        b, dp, hp = _coords(u)
        pltpu.sync_copy(x_hbm.at[b, dp, :, hp, :, :], xb_v)

        def do_row(pdph, c2):
            bv = [_bin16(xb_v[pdph // 4, pdph % 4, pl.ds(q * _L, _L)])
                  for q in range(6)]
            for m in range(3):
                b1, b2 = bv[2 * m], bv[2 * m + 1]
                c1 = jnp.take_along_axis(b1, pat_e, axis=0) * _L + \
                    jnp.take_along_axis(b1, pat_o, axis=0)
                c2_ = jnp.take_along_axis(b2, pat_e, axis=0) * _L + \
                    jnp.take_along_axis(b2, pat_o, axis=0)
                bp_v[pdph, pl.ds(m * _L, _L)] = c2_ + lo * (c1 - c2_)
            return c2

        lax.fori_loop(0, _P * _P, do_row, 0)

    def phase2(buf):
        def do_pdph(pdph, c3):
            off = pdph * 2 * _L
            for m in range(3):
                bpv = bp_v[pdph, pl.ds(m * _L, _L)]
                for l in range(_L):
                    p = m * _L + l
                    wp, j = p // 2, p % 2
                    out_v[buf, wp, pl.ds(j * _L + off, _L)] = vtp_v[bpv[l]]
            return c3

        lax.fori_loop(0, _P * _P, do_pdph, 0)

    def fire_out(u, buf):
        b, dp, hp = _coords(u)
        pltpu.async_copy(out_v.at[buf], out_hbm.at[b, dp, hp], sem_out[buf])

    def wait_out(u, buf):
        b, dp, hp = _coords(u)
        pltpu.make_async_copy(out_v.at[buf], out_hbm.at[b, dp, hp],
                              sem_out[buf]).wait()

    u0 = wid * _UPW

    def body(i, carry):
        for s in range(2):
            u = u0 + 2 * i + s
            phase1(u)

            @pl.when(2 * i + s >= 2)
            def _():
                wait_out(u - 2, s)

            phase2(s)
            fire_out(u, s)
        return carry

    lax.fori_loop(0, _UPW // 2, body, 0)
    wait_out(u0 + _UPW - 2, 0)
    wait_out(u0 + _UPW - 1, 1)


def _make_sc_call():
    mesh = plsc.VectorSubcoreMesh(core_axis_name="c", subcore_axis_name="s",
                                  num_cores=_NC, num_subcores=_NS)
    return pl.kernel(
        _tec_body,
        out_type=jax.ShapeDtypeStruct((2, _G, _G, _G, _CH), jnp.float32),
        mesh=mesh,
        scratch_types=[
            pltpu.VMEM((_P, _P, _W), jnp.float32),
            pltpu.VMEM((_NV * _L, _L), jnp.float32),
            pltpu.VMEM((_P * _P, 2 * _G), jnp.int32),
            pltpu.VMEM((2, _G, _CH), jnp.float32),
            pltpu.SemaphoreType.DMA,
            pltpu.SemaphoreType.DMA,
        ],
    )


@jax.jit
def kernel(x, vectors):
    B, C, D, H, W = x.shape
    x6 = x.reshape(B, D // _P, _P, H // _P, _P, W)
    va = jnp.broadcast_to(vectors[:, None, :], (_NV, _NV, _VD))
    vb = jnp.broadcast_to(vectors[None, :, :], (_NV, _NV, _VD))
    vtp = jnp.concatenate([va, vb], axis=-1)
    vtp = jnp.pad(vtp, ((0, 0), (0, _L - _NV), (0, 0)))
    vtp = vtp.reshape(_NV * _L, _L)
    out = _make_sc_call()(x6, vtp)
    return jnp.transpose(out, (0, 4, 1, 2, 3))
